# Initial kernel scaffold; baseline (speedup 1.0000x reference)
import jax
import jax.numpy as jnp
from jax.experimental import pallas as pl


def kernel(encoded_text, attention_mask, categorical_vars, emb_table, cat_tables, W, b):
    raise NotImplementedError("write your pallas kernel here")


# speedup vs baseline: 1.4820x; 1.4820x over previous
import functools

import jax
import jax.numpy as jnp
from jax import lax
from jax.experimental import pallas as pl
from jax.experimental.pallas import tpu as pltpu
from jax.experimental.pallas import tpu_sc as plsc

B, S, V, D = 4096, 200, 1000000, 64
NCF, CV, CD = 26, 100000, 16
NCLS = 1000

NUM_CORES, NUM_SUBCORES = 2, 16
NW = NUM_CORES * NUM_SUBCORES
BPW = B // NW
CPW = BPW * NCF
S0, S1 = 96, 104

_sc_mesh = plsc.VectorSubcoreMesh(core_axis_name="c", subcore_axis_name="s")


@functools.partial(
    pl.kernel,
    out_type=[
        jax.ShapeDtypeStruct((B, D), jnp.float32),
        jax.ShapeDtypeStruct((B * NCF, CD), jnp.float32),
    ],
    mesh=_sc_mesh,
    compiler_params=pltpu.CompilerParams(use_tc_tiling_on_sc=False),
    scratch_types=[
        pltpu.VMEM((BPW, S), jnp.int32),
        pltpu.VMEM((S, D), jnp.float32),
        pltpu.VMEM((BPW, D), jnp.float32),
        pltpu.VMEM((CPW,), jnp.int32),
        pltpu.VMEM((CPW,), jnp.int32),
        pltpu.VMEM((CPW, CD), jnp.float32),
        pltpu.SemaphoreType.DMA,
        pltpu.SemaphoreType.DMA,
    ],
)
def _sc_embed(tidx_hbm, emb_hbm, cvars_hbm, cat_hbm, xt_hbm, xc_hbm,
              tidx_v, rows_v, xt_v, cvars_v, cidx_v, xc_v, sem_t, sem_c):
    wid = lax.axis_index("s") * NUM_CORES + lax.axis_index("c")
    tbase = wid * BPW
    cbase = wid * CPW

    pltpu.sync_copy(tidx_hbm.at[pl.ds(tbase, BPW)], tidx_v)
    pltpu.sync_copy(cvars_hbm.at[pl.ds(cbase, CPW)], cvars_v)

    iota16 = lax.iota(jnp.int32, 16)

    def cidx_body(j, carry):
        p = pl.multiple_of(j * 16, 16)
        pv = p + iota16
        fld = pv % NCF
        cidx_v[pl.ds(p, 16)] = cvars_v[pl.ds(p, 16)] + fld * CV
        return carry

    lax.fori_loop(0, CPW // 16, cidx_body, 0)

    cat_handles = []
    for g in range(CPW // 128):
        cat_handles.append(pltpu.async_copy(
            cat_hbm.at[cidx_v.at[pl.ds(g * 128, 128)]],
            xc_v.at[pl.ds(g * 128, 128)],
            sem_c))

    def row_body(bi, carry):
        h1 = pltpu.async_copy(emb_hbm.at[tidx_v.at[bi, pl.ds(0, S0)]],
                              rows_v.at[pl.ds(0, S0)], sem_t)
        h2 = pltpu.async_copy(emb_hbm.at[tidx_v.at[bi, pl.ds(S0, S1)]],
                              rows_v.at[pl.ds(S0, S1)], sem_t)
        h1.wait()
        h2.wait()

        def red(s, acc):
            return tuple(acc[j] + rows_v[s, pl.ds(j * 16, 16)] for j in range(4))

        z = jnp.zeros((16,), jnp.float32)
        a = lax.fori_loop(0, S, red, (z, z, z, z))
        for j in range(4):
            xt_v[bi, pl.ds(j * 16, 16)] = a[j]
        return carry

    lax.fori_loop(0, BPW, row_body, 0)

    pltpu.sync_copy(xt_v, xt_hbm.at[pl.ds(tbase, BPW)])
    for h in cat_handles:
        h.wait()
    pltpu.sync_copy(xc_v, xc_hbm.at[pl.ds(cbase, CPW)])


def _tc_head(xt_ref, mask_ref, xc_ref, w1_ref, w2_ref, b_ref, o_ref):
    denom = jnp.clip(jnp.sum(mask_ref[...], axis=1, keepdims=True), 1.0, None)
    xt = xt_ref[...] / denom
    acc = lax.dot_general(xt, w1_ref[...], (((1,), (1,)), ((), ())),
                          preferred_element_type=jnp.float32)
    acc = acc + lax.dot_general(xc_ref[...], w2_ref[...], (((1,), (1,)), ((), ())),
                                preferred_element_type=jnp.float32)
    o_ref[...] = acc + b_ref[...]


BM, BN = 512, 128


def kernel(encoded_text, attention_mask, categorical_vars, emb_table, cat_tables, W, b):
    cvars_flat = categorical_vars.reshape(B * NCF)
    cat_flat = cat_tables.reshape(NCF * CV, CD)

    xt_sum, xc = _sc_embed(encoded_text, emb_table, cvars_flat, cat_flat)
    xc = xc.reshape(B, NCF * CD)

    w1 = W[:, :D]
    w2 = W[:, D:]
    b2 = b.reshape(1, NCLS)

    out = pl.pallas_call(
        _tc_head,
        grid=(B // BM, pl.cdiv(NCLS, BN)),
        in_specs=[
            pl.BlockSpec((BM, D), lambda i, j: (i, 0)),
            pl.BlockSpec((BM, S), lambda i, j: (i, 0)),
            pl.BlockSpec((BM, NCF * CD), lambda i, j: (i, 0)),
            pl.BlockSpec((BN, D), lambda i, j: (j, 0)),
            pl.BlockSpec((BN, NCF * CD), lambda i, j: (j, 0)),
            pl.BlockSpec((1, BN), lambda i, j: (0, j)),
        ],
        out_specs=pl.BlockSpec((BM, BN), lambda i, j: (i, j)),
        out_shape=jax.ShapeDtypeStruct((B, NCLS), jnp.float32),
    )(xt_sum, attention_mask, xc, w1, w2, b2)
    return out


# speedup vs baseline: 1.5909x; 1.0734x over previous
import functools

import jax
import jax.numpy as jnp
from jax import lax
from jax.experimental import pallas as pl
from jax.experimental.pallas import tpu as pltpu
from jax.experimental.pallas import tpu_sc as plsc

B, S, V, D = 4096, 200, 1000000, 64
NCF, CV, CD = 26, 100000, 16
NCLS = 1000

NUM_CORES, NUM_SUBCORES = 2, 16
NW = NUM_CORES * NUM_SUBCORES
BPW = B // NW
CPW = BPW * NCF
S0, S1 = 96, 104

_sc_mesh = plsc.VectorSubcoreMesh(core_axis_name="c", subcore_axis_name="s")


@functools.partial(
    pl.kernel,
    out_type=[
        jax.ShapeDtypeStruct((B, D), jnp.float32),
        jax.ShapeDtypeStruct((B * NCF, CD), jnp.float32),
    ],
    mesh=_sc_mesh,
    compiler_params=pltpu.CompilerParams(use_tc_tiling_on_sc=False),
    scratch_types=[
        pltpu.VMEM((BPW, S), jnp.int32),
        pltpu.VMEM((S, D), jnp.float32),
        pltpu.VMEM((S, D), jnp.float32),
        pltpu.VMEM((BPW, D), jnp.float32),
        pltpu.VMEM((CPW,), jnp.int32),
        pltpu.VMEM((CPW,), jnp.int32),
        pltpu.VMEM((CPW, CD), jnp.float32),
        pltpu.SemaphoreType.DMA,
        pltpu.SemaphoreType.DMA,
        pltpu.SemaphoreType.DMA,
    ],
)
def _sc_embed(tidx_hbm, emb_hbm, cvars_hbm, cat_hbm, xt_hbm, xc_hbm,
              tidx_v, buf_a, buf_b, xt_v, cvars_v, cidx_v, xc_v,
              sem_a, sem_b, sem_c):
    wid = lax.axis_index("s") * NUM_CORES + lax.axis_index("c")
    tbase = wid * BPW
    cbase = wid * CPW

    pltpu.sync_copy(tidx_hbm.at[pl.ds(tbase, BPW)], tidx_v)
    pltpu.sync_copy(cvars_hbm.at[pl.ds(cbase, CPW)], cvars_v)

    iota16 = lax.iota(jnp.int32, 16)

    def cidx_body(j, carry):
        p = pl.multiple_of(j * 16, 16)
        pv = p + iota16
        fld = pv % NCF
        cidx_v[pl.ds(p, 16)] = cvars_v[pl.ds(p, 16)] + fld * CV
        return carry

    lax.fori_loop(0, CPW // 16, cidx_body, 0)

    cat_handles = []
    for g in range(CPW // 128):
        cat_handles.append(pltpu.async_copy(
            cat_hbm.at[cidx_v.at[pl.ds(g * 128, 128)]],
            xc_v.at[pl.ds(g * 128, 128)],
            sem_c))

    def issue(buf, sem, r):
        pltpu.async_copy(emb_hbm.at[tidx_v.at[r, pl.ds(0, S0)]],
                         buf.at[pl.ds(0, S0)], sem)
        pltpu.async_copy(emb_hbm.at[tidx_v.at[r, pl.ds(S0, S1)]],
                         buf.at[pl.ds(S0, S1)], sem)

    def drain(buf, sem):
        pltpu.make_async_copy(emb_hbm.at[pl.ds(0, S0)],
                              buf.at[pl.ds(0, S0)], sem).wait()
        pltpu.make_async_copy(emb_hbm.at[pl.ds(0, S1)],
                              buf.at[pl.ds(0, S1)], sem).wait()

    RCHUNK, NCHUNK = 25, S // 25

    def reduce_into(buf, r):
        def chunk(c, tots):
            base = c * RCHUNK
            t = list(tots)
            for g in range(RCHUNK):
                for j in range(4):
                    t[j] = t[j] + buf[base + g, pl.ds(16 * j, 16)]
            return tuple(t)

        z = jnp.zeros((16,), jnp.float32)
        tots = lax.fori_loop(0, NCHUNK, chunk, (z, z, z, z))
        for j in range(4):
            xt_v[r, pl.ds(16 * j, 16)] = tots[j]

    issue(buf_a, sem_a, 0)
    issue(buf_b, sem_b, 1)

    def row_body(k, carry):
        r = 2 * k
        drain(buf_a, sem_a)
        reduce_into(buf_a, r)
        issue(buf_a, sem_a, r + 2)
        drain(buf_b, sem_b)
        reduce_into(buf_b, r + 1)
        issue(buf_b, sem_b, r + 3)
        return carry

    lax.fori_loop(0, BPW // 2 - 1, row_body, 0)
    drain(buf_a, sem_a)
    reduce_into(buf_a, BPW - 2)
    drain(buf_b, sem_b)
    reduce_into(buf_b, BPW - 1)

    pltpu.sync_copy(xt_v, xt_hbm.at[pl.ds(tbase, BPW)])
    for h in cat_handles:
        h.wait()
    pltpu.sync_copy(xc_v, xc_hbm.at[pl.ds(cbase, CPW)])


def _tc_head(xt_ref, mask_ref, xc_ref, w1_ref, w2_ref, b_ref, o_ref):
    denom = jnp.clip(jnp.sum(mask_ref[...], axis=1, keepdims=True), 1.0, None)
    xt = xt_ref[...] / denom
    acc = lax.dot_general(xt, w1_ref[...], (((1,), (1,)), ((), ())),
                          preferred_element_type=jnp.float32)
    acc = acc + lax.dot_general(xc_ref[...], w2_ref[...], (((1,), (1,)), ((), ())),
                                preferred_element_type=jnp.float32)
    o_ref[...] = acc + b_ref[...]


BM, BN = 512, 128


def kernel(encoded_text, attention_mask, categorical_vars, emb_table, cat_tables, W, b):
    cvars_flat = categorical_vars.reshape(B * NCF)
    cat_flat = cat_tables.reshape(NCF * CV, CD)

    xt_sum, xc = _sc_embed(encoded_text, emb_table, cvars_flat, cat_flat)
    xc = xc.reshape(B, NCF * CD)

    w1 = W[:, :D]
    w2 = W[:, D:]
    b2 = b.reshape(1, NCLS)

    out = pl.pallas_call(
        _tc_head,
        grid=(B // BM, pl.cdiv(NCLS, BN)),
        in_specs=[
            pl.BlockSpec((BM, D), lambda i, j: (i, 0)),
            pl.BlockSpec((BM, S), lambda i, j: (i, 0)),
            pl.BlockSpec((BM, NCF * CD), lambda i, j: (i, 0)),
            pl.BlockSpec((BN, D), lambda i, j: (j, 0)),
            pl.BlockSpec((BN, NCF * CD), lambda i, j: (j, 0)),
            pl.BlockSpec((1, BN), lambda i, j: (0, j)),
        ],
        out_specs=pl.BlockSpec((BM, BN), lambda i, j: (i, j)),
        out_shape=jax.ShapeDtypeStruct((B, NCLS), jnp.float32),
    )(xt_sum, attention_mask, xc, w1, w2, b2)
    return out


# speedup vs baseline: 1.5959x; 1.0031x over previous
import functools

import jax
import jax.numpy as jnp
from jax import lax
from jax.experimental import pallas as pl
from jax.experimental.pallas import tpu as pltpu
from jax.experimental.pallas import tpu_sc as plsc

B, S, V, D = 4096, 200, 1000000, 64
NCF, CV, CD = 26, 100000, 16
NCLS = 1000

NUM_CORES, NUM_SUBCORES = 2, 16
NW = NUM_CORES * NUM_SUBCORES
BPW = B // NW
S0, S1 = 96, 104

_sc_mesh = plsc.VectorSubcoreMesh(core_axis_name="c", subcore_axis_name="s")


@functools.partial(
    pl.kernel,
    out_type=[
        jax.ShapeDtypeStruct((B, D), jnp.float32),
        jax.ShapeDtypeStruct((NCF, B, CD), jnp.float32),
    ],
    mesh=_sc_mesh,
    compiler_params=pltpu.CompilerParams(use_tc_tiling_on_sc=False),
    scratch_types=[
        pltpu.VMEM((BPW, S), jnp.int32),
        pltpu.VMEM((S, D), jnp.float32),
        pltpu.VMEM((S, D), jnp.float32),
        pltpu.VMEM((BPW, D), jnp.float32),
        pltpu.VMEM((NCF, BPW), jnp.int32),
        pltpu.VMEM((NCF, BPW, CD), jnp.float32),
        pltpu.SemaphoreType.DMA,
        pltpu.SemaphoreType.DMA,
        pltpu.SemaphoreType.DMA,
    ],
)
def _sc_embed(tidx_hbm, emb_hbm, cvars_hbm, cat_hbm, xt_hbm, xc3_hbm,
              tidx_v, buf_a, buf_b, xt_v, cidx_v, xc_v,
              sem_a, sem_b, sem_c):
    wid = lax.axis_index("s") * NUM_CORES + lax.axis_index("c")
    tbase = wid * BPW

    pltpu.sync_copy(tidx_hbm.at[pl.ds(tbase, BPW)], tidx_v)

    pltpu.sync_copy(cvars_hbm.at[pl.ds(0, NCF), pl.ds(tbase, BPW)], cidx_v)

    cat_handles = []
    for c in range(NCF):
        cat_handles.append(pltpu.async_copy(
            cat_hbm.at[c].at[cidx_v.at[c]], xc_v.at[c], sem_c))

    def issue(buf, sem, r):
        pltpu.async_copy(emb_hbm.at[tidx_v.at[r, pl.ds(0, S0)]],
                         buf.at[pl.ds(0, S0)], sem)
        pltpu.async_copy(emb_hbm.at[tidx_v.at[r, pl.ds(S0, S1)]],
                         buf.at[pl.ds(S0, S1)], sem)

    def drain(buf, sem):
        pltpu.make_async_copy(emb_hbm.at[pl.ds(0, S0)],
                              buf.at[pl.ds(0, S0)], sem).wait()
        pltpu.make_async_copy(emb_hbm.at[pl.ds(0, S1)],
                              buf.at[pl.ds(S0, S1)], sem).wait()

    RCHUNK, NCHUNK = 25, S // 25

    def reduce_into(buf, r):
        def chunk(c, tots):
            base = c * RCHUNK
            t = list(tots)
            for g in range(RCHUNK):
                for j in range(4):
                    t[j] = t[j] + buf[base + g, pl.ds(16 * j, 16)]
            return tuple(t)

        z = jnp.zeros((16,), jnp.float32)
        tots = lax.fori_loop(0, NCHUNK, chunk, (z, z, z, z))
        for j in range(4):
            xt_v[r, pl.ds(16 * j, 16)] = tots[j]

    issue(buf_a, sem_a, 0)
    issue(buf_b, sem_b, 1)

    def row_body(k, carry):
        r = 2 * k
        drain(buf_a, sem_a)
        reduce_into(buf_a, r)
        issue(buf_a, sem_a, r + 2)
        drain(buf_b, sem_b)
        reduce_into(buf_b, r + 1)
        issue(buf_b, sem_b, r + 3)
        return carry

    lax.fori_loop(0, BPW // 2 - 1, row_body, 0)
    drain(buf_a, sem_a)
    reduce_into(buf_a, BPW - 2)
    drain(buf_b, sem_b)
    reduce_into(buf_b, BPW - 1)

    pltpu.sync_copy(xt_v, xt_hbm.at[pl.ds(tbase, BPW)])
    for h in cat_handles:
        h.wait()
    pltpu.sync_copy(xc_v, xc3_hbm.at[pl.ds(0, NCF), pl.ds(tbase, BPW)])


def _tc_head(xt_ref, mask_ref, x3_ref, w_ref, b_ref, o_ref):
    denom = jnp.clip(jnp.sum(mask_ref[...], axis=1, keepdims=True), 1.0, None)
    parts = [xt_ref[...] / denom] + [x3_ref[c] for c in range(NCF)]
    x = jnp.concatenate(parts, axis=1)
    acc = lax.dot_general(x, w_ref[...], (((1,), (1,)), ((), ())),
                          preferred_element_type=jnp.float32)
    o_ref[...] = acc + b_ref[...]


BM = 512


def kernel(encoded_text, attention_mask, categorical_vars, emb_table, cat_tables, W, b):
    xt_sum, xc3 = _sc_embed(encoded_text, emb_table, categorical_vars.T, cat_tables)
    b2 = b.reshape(1, NCLS)

    out = pl.pallas_call(
        _tc_head,
        grid=(B // BM,),
        in_specs=[
            pl.BlockSpec((BM, D), lambda i: (i, 0)),
            pl.BlockSpec((BM, S), lambda i: (i, 0)),
            pl.BlockSpec((NCF, BM, CD), lambda i: (0, i, 0)),
            pl.BlockSpec((NCLS, D + NCF * CD), lambda i: (0, 0)),
            pl.BlockSpec((1, NCLS), lambda i: (0, 0)),
        ],
        out_specs=pl.BlockSpec((BM, NCLS), lambda i: (i, 0)),
        out_shape=jax.ShapeDtypeStruct((B, NCLS), jnp.float32),
    )(xt_sum, attention_mask, xc3, W, b2)
    return out
